# Initial kernel scaffold; baseline (speedup 1.0000x reference)
import jax
import jax.numpy as jnp
from jax.experimental import pallas as pl


def kernel(x, edge_index, edge_type, W_emb, W_emb_loop, b_emb, W1, W1_loop, b1, W2, W2_loop, b2):
    raise NotImplementedError("write your pallas kernel here")


# speedup vs baseline: 3.2115x; 3.2115x over previous
import functools

import jax
import jax.numpy as jnp
from jax import lax
from jax.experimental import pallas as pl
from jax.experimental.pallas import tpu as pltpu
from jax.experimental.pallas import tpu_sc as plsc

N = 10000
E = 320000
R = 20
CH = 80
NS = 16
NC = 2
NW = NC * NS
EPW = E // NW
NCH = EPW // CH
RCH = N // CH


def _sc_aggregate(proj_flat, gidx2d, dst2d, d):
    mesh = plsc.VectorSubcoreMesh(
        core_axis_name="c", subcore_axis_name="s", num_cores=NC, num_subcores=NS
    )
    zeros_blk = jnp.zeros((CH, d), jnp.float32)

    @functools.partial(
        pl.kernel,
        out_type=jax.ShapeDtypeStruct((2, N, d), jnp.float32),
        mesh=mesh,
        scratch_types=[
            pltpu.VMEM((NCH, CH), jnp.int32),
            pltpu.VMEM((NCH, CH), jnp.int32),
            pltpu.VMEM((CH, d), jnp.float32),
            pltpu.VMEM_SHARED((N, d), jnp.float32),
            pltpu.SemaphoreType.DMA,
        ],
        compiler_params=pltpu.CompilerParams(use_tc_tiling_on_sc=False),
    )
    def k(proj_h, gidx_h, dst_h, zeros_h, out_h, gidx_v, dst_v, rows_v, agg_s, sem):
        c = lax.axis_index("c")
        s = lax.axis_index("s")
        w = c * NS + s
        pltpu.sync_copy(gidx_h.at[w], gidx_v)
        pltpu.sync_copy(dst_h.at[w], dst_v)

        def zbody(t, carry):
            j = s + t * NS

            @pl.when(j < RCH)
            def _():
                pltpu.sync_copy(zeros_h, agg_s.at[pl.ds(j * CH, CH)])

            return carry

        lax.fori_loop(0, (RCH + NS - 1) // NS, zbody, 0)
        plsc.subcore_barrier()

        def body(t, carry):
            pltpu.async_copy(proj_h.at[gidx_v.at[t]], rows_v, sem).wait()
            pltpu.sync_copy(rows_v, agg_s.at[dst_v.at[t]], add=True)
            return carry

        lax.fori_loop(0, NCH, body, 0)
        plsc.subcore_barrier()

        def dbody(t, carry):
            j = s + t * NS

            @pl.when(j < RCH)
            def _():
                pltpu.sync_copy(agg_s.at[pl.ds(j * CH, CH)], rows_v)
                pltpu.sync_copy(rows_v, out_h.at[c, pl.ds(j * CH, CH)])

            return carry

        lax.fori_loop(0, (RCH + NS - 1) // NS, dbody, 0)

    return k(proj_flat, gidx2d, dst2d, zeros_blk)


def _tc_proj(x, W2d, Wloop, b, d_in, d_out):
    BN = 1000

    def body(x_ref, w_ref, wl_ref, b_ref, proj_ref, loop_ref):
        xb = x_ref[...]
        proj_ref[...] = jnp.dot(xb, w_ref[...], preferred_element_type=jnp.float32)
        loop_ref[...] = (
            jnp.dot(xb, wl_ref[...], preferred_element_type=jnp.float32) + b_ref[...]
        )

    return pl.pallas_call(
        body,
        grid=(N // BN,),
        in_specs=[
            pl.BlockSpec((BN, d_in), lambda i: (i, 0)),
            pl.BlockSpec((d_in, R * d_out), lambda i: (0, 0)),
            pl.BlockSpec((d_in, d_out), lambda i: (0, 0)),
            pl.BlockSpec((1, d_out), lambda i: (0, 0)),
        ],
        out_specs=[
            pl.BlockSpec((BN, R * d_out), lambda i: (i, 0)),
            pl.BlockSpec((BN, d_out), lambda i: (i, 0)),
        ],
        out_shape=[
            jax.ShapeDtypeStruct((N, R * d_out), jnp.float32),
            jax.ShapeDtypeStruct((N, d_out), jnp.float32),
        ],
    )(x, W2d, Wloop, b.reshape(1, d_out))


def _tc_combine(parts, loop, relu, d):
    BN = 1000

    def body(p_ref, l_ref, o_ref):
        h = p_ref[0] + p_ref[1] + l_ref[...]
        if relu:
            h = jnp.maximum(h, 0.0)
        o_ref[...] = h

    return pl.pallas_call(
        body,
        grid=(N // BN,),
        in_specs=[
            pl.BlockSpec((2, BN, d), lambda i: (0, i, 0)),
            pl.BlockSpec((BN, d), lambda i: (i, 0)),
        ],
        out_specs=pl.BlockSpec((BN, d), lambda i: (i, 0)),
        out_shape=jax.ShapeDtypeStruct((N, d), jnp.float32),
    )(parts, loop)


def kernel(x, edge_index, edge_type, W_emb, W_emb_loop, b_emb,
           W1, W1_loop, b1, W2, W2_loop, b2):
    src = edge_index[0]
    dst = edge_index[1]
    gidx2d = (src * R + edge_type).astype(jnp.int32).reshape(NW, NCH, CH)
    dst2d = dst.astype(jnp.int32).reshape(NW, NCH, CH)

    h = x
    for Wr, Wl, b, relu in (
        (W_emb, W_emb_loop, b_emb, True),
        (W1, W1_loop, b1, True),
        (W2, W2_loop, b2, False),
    ):
        d_in, d_out = Wr.shape[1], Wr.shape[2]
        W2d = Wr.transpose(1, 0, 2).reshape(d_in, R * d_out)
        proj, loop = _tc_proj(h, W2d, Wl, b, d_in, d_out)
        parts = _sc_aggregate(proj.reshape(N * R, d_out), gidx2d, dst2d, d_out)
        h = _tc_combine(parts, loop, relu, d_out)
    return h
